# baseline (speedup 1.0000x reference)
import functools
import jax
import jax.numpy as jnp
from jax import lax
from jax.experimental import pallas as pl
from jax.experimental.pallas import tpu as pltpu
from jax.experimental.pallas import tpu_sc as plsc

_N = 10000
_E = 320000
_HE = 10000
_G = 64
_F = 128

_NC = 2
_NS = 16
_W = 40
_EPT = _E // (_NC * _NS)
_NWIN = _EPT // _W
_NP = 10240
_RPT = _NP // _NS

_CW = 80
_CNWIN = _EPT // _CW
_CPAD = 10240
_CPT = _CPAD // _NS


def _sc_mesh():
    return plsc.VectorSubcoreMesh(
        core_axis_name="c", subcore_axis_name="s",
        num_cores=_NC, num_subcores=_NS)


def _phase_body(table, src, dst, out, *refs):
    IS = refs[0:8]
    ID = refs[8:16]
    RW = refs[16:24]
    accum, sem_i, sem_g, sem_s = refs[24:]
    c = lax.axis_index("c")
    s = lax.axis_index("s")
    tile = c * _NS + s
    ebase = tile * _EPT
    row0 = s * _RPT

    zeros16 = jnp.zeros((16,), jnp.float32)

    def zrow(r, carry):
        for j in range(8):
            RW[0][r, pl.ds(j * 16, 16)] = zeros16
        return carry
    lax.fori_loop(0, _W, zrow, 0)
    for k in range(_RPT // _W):
        pltpu.sync_copy(RW[0], accum.at[pl.ds(row0 + k * _W, _W)])
    plsc.subcore_barrier()

    def issue_idx(w, r):
        base = ebase + w * _W
        pltpu.async_copy(src.at[pl.ds(base, _W)], IS[r], sem_i)
        pltpu.async_copy(dst.at[pl.ds(base, _W)], ID[r], sem_i)

    def wait_idx(r):
        pltpu.make_async_copy(src.at[pl.ds(0, _W)], IS[r], sem_i).wait()
        pltpu.make_async_copy(dst.at[pl.ds(0, _W)], ID[r], sem_i).wait()

    def issue_gather(r):
        pltpu.async_copy(table.at[IS[r]], RW[r], sem_g)

    def wait_gather(r):
        pltpu.make_async_copy(table.at[IS[r]], RW[r], sem_g).wait()

    def issue_scatter(r):
        pltpu.async_copy(RW[r], accum.at[ID[r]], sem_s, add=True)

    def wait_scatter(r):
        pltpu.make_async_copy(RW[r], accum.at[ID[r]], sem_s).wait()

    def step(w, r, sc_wait, guard_idx, do_idx, do_gather):
        wait_gather(r)
        issue_scatter(r)
        if sc_wait:
            wait_scatter(r)
        if do_idx:
            if guard_idx:
                @pl.when(w + 4 < _NWIN)
                def _():
                    issue_idx(w + 4, (r + 4) % 8)
            else:
                issue_idx(w + 4, (r + 4) % 8)
        if do_gather:
            wait_idx((r + 3) % 8)
            issue_gather((r + 3) % 8)

    for w0 in range(4):
        issue_idx(w0, w0)
    for w0 in range(3):
        wait_idx(w0)
        issue_gather(w0)
    step(0, 0, False, False, True, True)
    step(1, 1, False, False, True, True)
    step(2, 2, False, False, True, True)
    step(3, 3, True, False, True, True)

    def iter8(i, carry):
        w = 4 + 8 * i
        for k in range(8):
            step(w + k, (4 + k) % 8, True, False, True, True)
        return carry
    lax.fori_loop(0, (_NWIN - 10) // 8, iter8, 0)

    step(244, 4, True, False, True, True)
    step(245, 5, True, False, True, True)
    step(246, 6, True, False, False, True)
    step(247, 7, True, False, False, False)
    step(248, 0, True, False, False, False)
    step(249, 1, True, False, False, False)
    wait_scatter(7)
    wait_scatter(0)
    wait_scatter(1)

    plsc.subcore_barrier()
    pltpu.sync_copy(accum.at[pl.ds(row0, _RPT)], out.at[c, pl.ds(row0, _RPT)])


@jax.jit
def _sc_phase(table, src, dst):
    return pl.kernel(
        _phase_body,
        out_type=jax.ShapeDtypeStruct((_NC, _NP, _F), jnp.float32),
        mesh=_sc_mesh(),
        scratch_types=(
            [pltpu.VMEM((_W,), jnp.int32)] * 8
            + [pltpu.VMEM((_W,), jnp.int32)] * 8
            + [pltpu.VMEM((_W, _F), jnp.float32)] * 8
            + [pltpu.VMEM_SHARED((_NP, _F), jnp.float32),
               pltpu.SemaphoreType.DMA, pltpu.SemaphoreType.DMA,
               pltpu.SemaphoreType.DMA]
        ),
    )(table, src, dst)


def _count_body(src, dst, out, zbuf, onesb, idxs0, idxd0, idxs1, idxd1,
                idxs2, idxd2, dacc, bacc, sem_i, sem_s):
    c = lax.axis_index("c")
    s = lax.axis_index("s")
    tile = c * _NS + s
    ebase = tile * _EPT
    row0 = s * _CPT

    zeros16 = jnp.zeros((16,), jnp.float32)
    ones16 = jnp.ones((16,), jnp.float32)
    for j in range(_CPT // 16):
        zbuf[pl.ds(j * 16, 16)] = zeros16
    for j in range(_CW // 16):
        onesb[pl.ds(j * 16, 16)] = ones16
    pltpu.sync_copy(zbuf, dacc.at[pl.ds(row0, _CPT)])
    pltpu.sync_copy(zbuf, bacc.at[pl.ds(row0, _CPT)])
    plsc.subcore_barrier()

    bufs = ((idxs0, idxd0), (idxs1, idxd1), (idxs2, idxd2))

    def issue_idx(w, r):
        base = ebase + w * _CW
        pltpu.async_copy(src.at[pl.ds(base, _CW)], bufs[r][0], sem_i)
        pltpu.async_copy(dst.at[pl.ds(base, _CW)], bufs[r][1], sem_i)

    def wait_idx(r):
        pltpu.make_async_copy(src.at[pl.ds(0, _CW)], bufs[r][0], sem_i).wait()
        pltpu.make_async_copy(dst.at[pl.ds(0, _CW)], bufs[r][1], sem_i).wait()

    def issue_sc(r):
        pltpu.async_copy(onesb, dacc.at[bufs[r][0]], sem_s, add=True)
        pltpu.async_copy(onesb, bacc.at[bufs[r][1]], sem_s, add=True)

    def wait_sc(r):
        pltpu.make_async_copy(onesb, dacc.at[bufs[r][0]], sem_s).wait()
        pltpu.make_async_copy(onesb, bacc.at[bufs[r][1]], sem_s).wait()

    def step(w, r, first, guard_idx):
        rp = (r + 2) % 3
        rn = (r + 1) % 3
        wait_idx(r)
        if not first:
            wait_sc(rp)
        issue_sc(r)
        if guard_idx:
            @pl.when(w + 2 < _CNWIN)
            def _():
                issue_idx(w + 2, rp)
        else:
            issue_idx(w + 2, rp)
        _ = rn

    issue_idx(0, 0)
    issue_idx(1, 1)
    step(0, 0, True, False)

    def iter3(i, carry):
        w = 1 + 3 * i
        step(w, 1, False, False)
        step(w + 1, 2, False, False)
        step(w + 2, 0, False, True)
        return carry
    lax.fori_loop(0, (_CNWIN - 2) // 3, iter3, 0)

    wait_idx(1)
    wait_sc(0)
    issue_sc(1)
    wait_sc(1)

    plsc.subcore_barrier()
    pltpu.sync_copy(dacc.at[pl.ds(row0, _CPT)], out.at[c, 0, pl.ds(row0, _CPT)])
    pltpu.sync_copy(bacc.at[pl.ds(row0, _CPT)], out.at[c, 1, pl.ds(row0, _CPT)])


@jax.jit
def _sc_count(src, dst):
    return pl.kernel(
        _count_body,
        out_type=jax.ShapeDtypeStruct((_NC, 2, _CPAD), jnp.float32),
        mesh=_sc_mesh(),
        scratch_types=(
            [pltpu.VMEM((_CPT,), jnp.float32),
             pltpu.VMEM((_CW,), jnp.float32)]
            + [pltpu.VMEM((_CW,), jnp.int32)] * 6
            + [pltpu.VMEM_SHARED((_CPAD,), jnp.float32),
               pltpu.VMEM_SHARED((_CPAD,), jnp.float32),
               pltpu.SemaphoreType.DMA, pltpu.SemaphoreType.DMA]
        ),
    )(src, dst)


_BLK = 1000
_NBLK = _N // _BLK


def _mm_plain_body(x_ref, w_ref, o_ref):
    o_ref[...] = jnp.dot(x_ref[...], w_ref[...],
                         preferred_element_type=jnp.float32)


@jax.jit
def _tc_mm_plain(x, w):
    return pl.pallas_call(
        _mm_plain_body,
        grid=(_NBLK,),
        in_specs=[
            pl.BlockSpec((_BLK, 128), lambda i: (i, 0)),
            pl.BlockSpec((128, 128), lambda i: (0, 0)),
        ],
        out_specs=pl.BlockSpec((_BLK, 128), lambda i: (i, 0)),
        out_shape=jax.ShapeDtypeStruct((_N, 128), jnp.float32),
    )(x, w)


def _combine_body(p_ref, s_ref, o_ref):
    acc = p_ref[0] + p_ref[1]
    o_ref[...] = s_ref[...] * acc


@jax.jit
def _tc_combine(p, scale):
    return pl.pallas_call(
        _combine_body,
        grid=(_NBLK,),
        in_specs=[
            pl.BlockSpec((_NC, _BLK, _F), lambda i: (0, i, 0)),
            pl.BlockSpec((_BLK, 1), lambda i: (i, 0)),
        ],
        out_specs=pl.BlockSpec((_BLK, _F), lambda i: (i, 0)),
        out_shape=jax.ShapeDtypeStruct((_N, _F), jnp.float32),
    )(p, scale)


def _mm_fused1_body(p_ref, s_ref, b_ref, w_ref, o1_ref, o2_ref):
    h = s_ref[...] * (p_ref[0] + p_ref[1]) + b_ref[...][None, :]
    h = jnp.maximum(h, 0.0)
    o = jnp.dot(h, w_ref[...], preferred_element_type=jnp.float32)
    o1_ref[...] = o[:, :128]
    o2_ref[...] = o[:, 128:]


@jax.jit
def _tc_mm_fused1(p, scale, b, w):
    return pl.pallas_call(
        _mm_fused1_body,
        grid=(_NBLK,),
        in_specs=[
            pl.BlockSpec((_NC, _BLK, _F), lambda i: (0, i, 0)),
            pl.BlockSpec((_BLK, 1), lambda i: (i, 0)),
            pl.BlockSpec((128,), lambda i: (0,)),
            pl.BlockSpec((128, 256), lambda i: (0, 0)),
        ],
        out_specs=[
            pl.BlockSpec((_BLK, 128), lambda i: (i, 0)),
            pl.BlockSpec((_BLK, 128), lambda i: (i, 0)),
        ],
        out_shape=[
            jax.ShapeDtypeStruct((_N, 128), jnp.float32),
            jax.ShapeDtypeStruct((_N, 128), jnp.float32),
        ],
    )(p, scale, b, w)


def _mm_fused2_body(pa_ref, pb_ref, s_ref, ba_ref, bb_ref, wa_ref, wb_ref,
                    o_ref):
    sc = s_ref[...]
    ha = jnp.maximum(sc * (pa_ref[0] + pa_ref[1]) + ba_ref[...][None, :], 0.0)
    hb = jnp.maximum(sc * (pb_ref[0] + pb_ref[1]) + bb_ref[...][None, :], 0.0)
    o_ref[...] = (jnp.dot(ha, wa_ref[...], preferred_element_type=jnp.float32)
                  + jnp.dot(hb, wb_ref[...], preferred_element_type=jnp.float32))


@jax.jit
def _tc_mm_fused2(pa, pb, scale, b_lo, b_hi, w_lo, w_hi):
    return pl.pallas_call(
        _mm_fused2_body,
        grid=(_NBLK,),
        in_specs=[
            pl.BlockSpec((_NC, _BLK, _F), lambda i: (0, i, 0)),
            pl.BlockSpec((_NC, _BLK, _F), lambda i: (0, i, 0)),
            pl.BlockSpec((_BLK, 1), lambda i: (i, 0)),
            pl.BlockSpec((128,), lambda i: (0,)),
            pl.BlockSpec((128,), lambda i: (0,)),
            pl.BlockSpec((128, 128), lambda i: (0, 0)),
            pl.BlockSpec((128, 128), lambda i: (0, 0)),
        ],
        out_specs=pl.BlockSpec((_BLK, 128), lambda i: (i, 0)),
        out_shape=jax.ShapeDtypeStruct((_N, 128), jnp.float32),
    )(pa, pb, scale, b_lo, b_hi, w_lo, w_hi)


_FBLK = 1000
_NFBLK = _N // _FBLK


def _final_body(p_ref, s_ref, b_ref, bat_ref, lw1_ref, lb1_ref, lw2_ref,
                lb2_ref, o_ref, hacc, cacc):
    i = pl.program_id(0)

    @pl.when(i == 0)
    def _():
        hacc[...] = jnp.zeros_like(hacc)
        cacc[...] = jnp.zeros_like(cacc)

    h = s_ref[...] * (p_ref[0] + p_ref[1]) + b_ref[...][None, :]
    h = jnp.maximum(h, 0.0)
    seg = (lax.broadcasted_iota(jnp.int32, (_G, _FBLK), 0)
           == bat_ref[...].reshape(1, _FBLK)).astype(jnp.float32)
    hacc[...] += jnp.dot(seg, h, preferred_element_type=jnp.float32)
    cnt = jnp.sum(seg, axis=1)
    cacc[...] += jnp.broadcast_to(cnt[:, None], (_G, 128))

    @pl.when(i == _NFBLK - 1)
    def _():
        pooled = hacc[...] / jnp.maximum(cacc[...], 1.0)
        z = jnp.maximum(
            jnp.dot(pooled, lw1_ref[...], preferred_element_type=jnp.float32)
            + lb1_ref[...][None, :], 0.0)
        res = jnp.dot(z, lw2_ref[...], preferred_element_type=jnp.float32)
        o_ref[...] = jnp.broadcast_to(res + lb2_ref[...][None, :], (_G, 128))


@jax.jit
def _tc_final(p, scale, b, batch, lw1, lb1, lw2, lb2):
    return pl.pallas_call(
        _final_body,
        grid=(_NFBLK,),
        in_specs=[
            pl.BlockSpec((_NC, _FBLK, _F), lambda i: (0, i, 0)),
            pl.BlockSpec((_FBLK, 1), lambda i: (i, 0)),
            pl.BlockSpec((128,), lambda i: (0,)),
            pl.BlockSpec((_FBLK, 1), lambda i: (i, 0)),
            pl.BlockSpec((128, 64), lambda i: (0, 0)),
            pl.BlockSpec((64,), lambda i: (0,)),
            pl.BlockSpec((64, 1), lambda i: (0, 0)),
            pl.BlockSpec((1,), lambda i: (0,)),
        ],
        out_specs=pl.BlockSpec((_G, 128), lambda i: (0, 0)),
        out_shape=jax.ShapeDtypeStruct((_G, 128), jnp.float32),
        scratch_shapes=[
            pltpu.VMEM((_G, 128), jnp.float32),
            pltpu.VMEM((_G, 128), jnp.float32),
        ],
    )(p, scale, b, batch, lw1, lb1, lw2, lb2)


def kernel(x, hyperedge_index, batch, W1, b1, W2, b2, W3, b3,
           lw1, lb1, lw2, lb2):
    ei0 = hyperedge_index[0]
    ei1 = hyperedge_index[1]

    cnt = _sc_count(ei0, ei1)
    csum = cnt[0] + cnt[1]
    d = csum[0, :_N]
    bb = csum[1, :_HE]
    dinv = jnp.where(d > 0, 1.0 / d, 0.0).reshape(_N, 1)
    binv = jnp.where(bb > 0, 1.0 / bb, 0.0).reshape(_HE, 1)

    xl1 = _tc_mm_plain(x, W1)
    oe1 = _tc_combine(_sc_phase(xl1, ei0, ei1), binv)
    pb1 = _sc_phase(oe1, ei1, ei0)

    xl2a, xl2b = _tc_mm_fused1(pb1, dinv, b1, W2)
    oe2a = _tc_combine(_sc_phase(xl2a, ei0, ei1), binv)
    oe2b = _tc_combine(_sc_phase(xl2b, ei0, ei1), binv)
    pb2a = _sc_phase(oe2a, ei1, ei0)
    pb2b = _sc_phase(oe2b, ei1, ei0)

    xl3 = _tc_mm_fused2(pb2a, pb2b, dinv, b2[:128], b2[128:],
                        W3[:128], W3[128:])
    oe3 = _tc_combine(_sc_phase(xl3, ei0, ei1), binv)
    pb3 = _sc_phase(oe3, ei1, ei0)

    out = _tc_final(pb3, dinv, b3, batch.reshape(_N, 1), lw1, lb1, lw2, lb2)
    return out[:, :1]


# speedup vs baseline: 1.1214x; 1.1214x over previous
import functools
import jax
import jax.numpy as jnp
from jax import lax
from jax.experimental import pallas as pl
from jax.experimental.pallas import tpu as pltpu
from jax.experimental.pallas import tpu_sc as plsc

_N = 10000
_E = 320000
_HE = 10000
_G = 64
_F = 128

_NC = 2
_NS = 16
_W = 80
_EPT = _E // (_NC * _NS)
_NWIN = _EPT // _W
_NP = 10240
_RPT = _NP // _NS
_ZR = 64

_CPAD = 10240
_CPT = _CPAD // _NS


def _sc_mesh():
    return plsc.VectorSubcoreMesh(
        core_axis_name="c", subcore_axis_name="s",
        num_cores=_NC, num_subcores=_NS)


def _phase_body(table, src, dst, out, *refs):
    (is0, is1, is2, is3, id0, id1, id2, id3, id4,
     r0_, r1_, r2_, r3_, accum, sem_i, sem_g, sem_s) = refs
    IS = (is0, is1, is2, is3)
    ID = (id0, id1, id2, id3, id4)
    RW = (r0_, r1_, r2_, r3_)
    c = lax.axis_index("c")
    s = lax.axis_index("s")
    tile = c * _NS + s
    ebase = tile * _EPT
    row0 = s * _RPT

    zeros16 = jnp.zeros((16,), jnp.float32)

    def zrow(r, carry):
        for j in range(8):
            RW[0][r, pl.ds(j * 16, 16)] = zeros16
        return carry
    lax.fori_loop(0, _W, zrow, 0)
    for k in range(_RPT // _W):
        pltpu.sync_copy(RW[0], accum.at[pl.ds(row0 + k * _W, _W)])
    plsc.subcore_barrier()

    def issue_idx(w, s4, s5):
        base = ebase + w * _W
        pltpu.async_copy(src.at[pl.ds(base, _W)], IS[s4], sem_i)
        pltpu.async_copy(dst.at[pl.ds(base, _W)], ID[s5], sem_i)

    def wait_idx(s4, s5):
        pltpu.make_async_copy(src.at[pl.ds(0, _W)], IS[s4], sem_i).wait()
        pltpu.make_async_copy(dst.at[pl.ds(0, _W)], ID[s5], sem_i).wait()

    def issue_gather(s4):
        pltpu.async_copy(table.at[IS[s4]], RW[s4], sem_g)

    def wait_gather(s4):
        pltpu.make_async_copy(table.at[IS[s4]], RW[s4], sem_g).wait()

    def issue_scatter(s4, s5):
        pltpu.async_copy(RW[s4], accum.at[ID[s5]], sem_s, add=True)

    def wait_scatter(s4, s5):
        pltpu.make_async_copy(RW[s4], accum.at[ID[s5]], sem_s).wait()

    def step(w, s4, s5, sc_wait, guard_idx, do_idx, do_gather):
        wait_gather(s4)
        issue_scatter(s4, s5)
        if sc_wait:
            wait_scatter(s4, s5)
        if do_idx:
            if guard_idx:
                @pl.when(w + 3 < _NWIN)
                def _():
                    issue_idx(w + 3, (s4 + 3) % 4, (s5 + 3) % 5)
            else:
                issue_idx(w + 3, (s4 + 3) % 4, (s5 + 3) % 5)
        if do_gather:
            wait_idx((s4 + 2) % 4, (s5 + 2) % 5)
            issue_gather((s4 + 2) % 4)

    issue_idx(0, 0, 0)
    issue_idx(1, 1, 1)
    issue_idx(2, 2, 2)
    wait_idx(0, 0)
    issue_gather(0)
    wait_idx(1, 1)
    issue_gather(1)
    step(0, 0, 0, False, False, True, True)
    step(1, 1, 1, False, False, True, True)
    step(2, 2, 2, True, False, True, True)

    def iter20(i, carry):
        w = 3 + 20 * i
        for k in range(20):
            step(w + k, (3 + k) % 4, (3 + k) % 5, True, k == 19, True, True)
        return carry
    lax.fori_loop(0, (_NWIN - 5) // 20, iter20, 0)

    step(123, 3, 3, True, False, False, False)
    step(124, 0, 4, True, False, False, False)
    wait_scatter(3, 3)
    wait_scatter(0, 4)

    plsc.subcore_barrier()
    for k in range(_RPT // _W):
        r0 = row0 + k * _W
        pltpu.sync_copy(accum.at[pl.ds(r0, _W)], out.at[c, pl.ds(r0, _W)])


@jax.jit
def _sc_phase(table, src, dst):
    return pl.kernel(
        _phase_body,
        out_type=jax.ShapeDtypeStruct((_NC, _NP, _F), jnp.float32),
        mesh=_sc_mesh(),
        scratch_types=(
            [pltpu.VMEM((_W,), jnp.int32)] * 4
            + [pltpu.VMEM((_W,), jnp.int32)] * 5
            + [pltpu.VMEM((_W, _F), jnp.float32)] * 4
            + [pltpu.VMEM_SHARED((_NP, _F), jnp.float32),
               pltpu.SemaphoreType.DMA, pltpu.SemaphoreType.DMA,
               pltpu.SemaphoreType.DMA]
        ),
    )(table, src, dst)


def _count_body(src, dst, out, zbuf, onesb, idxs0, idxd0, idxs1, idxd1,
                idxs2, idxd2, dacc, bacc, sem_i, sem_s):
    c = lax.axis_index("c")
    s = lax.axis_index("s")
    tile = c * _NS + s
    ebase = tile * _EPT
    row0 = s * _CPT

    zeros16 = jnp.zeros((16,), jnp.float32)
    ones16 = jnp.ones((16,), jnp.float32)
    for j in range(_CPT // 16):
        zbuf[pl.ds(j * 16, 16)] = zeros16
    for j in range(_W // 16):
        onesb[pl.ds(j * 16, 16)] = ones16
    pltpu.sync_copy(zbuf, dacc.at[pl.ds(row0, _CPT)])
    pltpu.sync_copy(zbuf, bacc.at[pl.ds(row0, _CPT)])
    plsc.subcore_barrier()

    bufs = ((idxs0, idxd0), (idxs1, idxd1), (idxs2, idxd2))

    def issue_idx(w, r):
        base = ebase + w * _W
        pltpu.async_copy(src.at[pl.ds(base, _W)], bufs[r][0], sem_i)
        pltpu.async_copy(dst.at[pl.ds(base, _W)], bufs[r][1], sem_i)

    def wait_idx(r):
        pltpu.make_async_copy(src.at[pl.ds(0, _W)], bufs[r][0], sem_i).wait()
        pltpu.make_async_copy(dst.at[pl.ds(0, _W)], bufs[r][1], sem_i).wait()

    def issue_sc(r):
        pltpu.async_copy(onesb, dacc.at[bufs[r][0]], sem_s, add=True)
        pltpu.async_copy(onesb, bacc.at[bufs[r][1]], sem_s, add=True)

    def wait_sc(r):
        pltpu.make_async_copy(onesb, dacc.at[bufs[r][0]], sem_s).wait()
        pltpu.make_async_copy(onesb, bacc.at[bufs[r][1]], sem_s).wait()

    def step(w, r, first, guard_idx):
        rp = (r + 2) % 3
        rn = (r + 1) % 3
        wait_idx(r)
        if not first:
            wait_sc(rp)
        issue_sc(r)
        if guard_idx:
            @pl.when(w + 2 < _NWIN)
            def _():
                issue_idx(w + 2, rp)
        else:
            issue_idx(w + 2, rp)
        _ = rn

    issue_idx(0, 0)
    issue_idx(1, 1)
    step(0, 0, True, False)

    def iter3(i, carry):
        w = 1 + 3 * i
        step(w, 1, False, False)
        step(w + 1, 2, False, False)
        step(w + 2, 0, False, True)
        return carry
    lax.fori_loop(0, (_NWIN - 2) // 3, iter3, 0)

    wait_idx(1)
    wait_sc(0)
    issue_sc(1)
    wait_sc(1)

    plsc.subcore_barrier()
    pltpu.sync_copy(dacc.at[pl.ds(row0, _CPT)], out.at[c, 0, pl.ds(row0, _CPT)])
    pltpu.sync_copy(bacc.at[pl.ds(row0, _CPT)], out.at[c, 1, pl.ds(row0, _CPT)])


@jax.jit
def _sc_count(src, dst):
    return pl.kernel(
        _count_body,
        out_type=jax.ShapeDtypeStruct((_NC, 2, _CPAD), jnp.float32),
        mesh=_sc_mesh(),
        scratch_types=(
            [pltpu.VMEM((_CPT,), jnp.float32),
             pltpu.VMEM((_W,), jnp.float32)]
            + [pltpu.VMEM((_W,), jnp.int32)] * 6
            + [pltpu.VMEM_SHARED((_CPAD,), jnp.float32),
               pltpu.VMEM_SHARED((_CPAD,), jnp.float32),
               pltpu.SemaphoreType.DMA, pltpu.SemaphoreType.DMA]
        ),
    )(src, dst)


_BLK = 1000
_NBLK = _N // _BLK


def _mm_plain_body(x_ref, w_ref, o_ref):
    o_ref[...] = jnp.dot(x_ref[...], w_ref[...],
                         preferred_element_type=jnp.float32)


@jax.jit
def _tc_mm_plain(x, w):
    return pl.pallas_call(
        _mm_plain_body,
        grid=(_NBLK,),
        in_specs=[
            pl.BlockSpec((_BLK, 128), lambda i: (i, 0)),
            pl.BlockSpec((128, 128), lambda i: (0, 0)),
        ],
        out_specs=pl.BlockSpec((_BLK, 128), lambda i: (i, 0)),
        out_shape=jax.ShapeDtypeStruct((_N, 128), jnp.float32),
    )(x, w)


def _combine_body(p_ref, s_ref, o_ref):
    acc = p_ref[0] + p_ref[1]
    o_ref[...] = s_ref[...] * acc


@jax.jit
def _tc_combine(p, scale):
    return pl.pallas_call(
        _combine_body,
        grid=(_NBLK,),
        in_specs=[
            pl.BlockSpec((_NC, _BLK, _F), lambda i: (0, i, 0)),
            pl.BlockSpec((_BLK, 1), lambda i: (i, 0)),
        ],
        out_specs=pl.BlockSpec((_BLK, _F), lambda i: (i, 0)),
        out_shape=jax.ShapeDtypeStruct((_N, _F), jnp.float32),
    )(p, scale)


def _mm_fused1_body(p_ref, s_ref, b_ref, w_ref, o1_ref, o2_ref):
    h = s_ref[...] * (p_ref[0] + p_ref[1]) + b_ref[...][None, :]
    h = jnp.maximum(h, 0.0)
    o = jnp.dot(h, w_ref[...], preferred_element_type=jnp.float32)
    o1_ref[...] = o[:, :128]
    o2_ref[...] = o[:, 128:]


@jax.jit
def _tc_mm_fused1(p, scale, b, w):
    return pl.pallas_call(
        _mm_fused1_body,
        grid=(_NBLK,),
        in_specs=[
            pl.BlockSpec((_NC, _BLK, _F), lambda i: (0, i, 0)),
            pl.BlockSpec((_BLK, 1), lambda i: (i, 0)),
            pl.BlockSpec((128,), lambda i: (0,)),
            pl.BlockSpec((128, 256), lambda i: (0, 0)),
        ],
        out_specs=[
            pl.BlockSpec((_BLK, 128), lambda i: (i, 0)),
            pl.BlockSpec((_BLK, 128), lambda i: (i, 0)),
        ],
        out_shape=[
            jax.ShapeDtypeStruct((_N, 128), jnp.float32),
            jax.ShapeDtypeStruct((_N, 128), jnp.float32),
        ],
    )(p, scale, b, w)


def _mm_fused2_body(pa_ref, pb_ref, s_ref, ba_ref, bb_ref, wa_ref, wb_ref,
                    o_ref):
    sc = s_ref[...]
    ha = jnp.maximum(sc * (pa_ref[0] + pa_ref[1]) + ba_ref[...][None, :], 0.0)
    hb = jnp.maximum(sc * (pb_ref[0] + pb_ref[1]) + bb_ref[...][None, :], 0.0)
    o_ref[...] = (jnp.dot(ha, wa_ref[...], preferred_element_type=jnp.float32)
                  + jnp.dot(hb, wb_ref[...], preferred_element_type=jnp.float32))


@jax.jit
def _tc_mm_fused2(pa, pb, scale, b_lo, b_hi, w_lo, w_hi):
    return pl.pallas_call(
        _mm_fused2_body,
        grid=(_NBLK,),
        in_specs=[
            pl.BlockSpec((_NC, _BLK, _F), lambda i: (0, i, 0)),
            pl.BlockSpec((_NC, _BLK, _F), lambda i: (0, i, 0)),
            pl.BlockSpec((_BLK, 1), lambda i: (i, 0)),
            pl.BlockSpec((128,), lambda i: (0,)),
            pl.BlockSpec((128,), lambda i: (0,)),
            pl.BlockSpec((128, 128), lambda i: (0, 0)),
            pl.BlockSpec((128, 128), lambda i: (0, 0)),
        ],
        out_specs=pl.BlockSpec((_BLK, 128), lambda i: (i, 0)),
        out_shape=jax.ShapeDtypeStruct((_N, 128), jnp.float32),
    )(pa, pb, scale, b_lo, b_hi, w_lo, w_hi)


_FBLK = 1000
_NFBLK = _N // _FBLK


def _final_body(p_ref, s_ref, b_ref, bat_ref, lw1_ref, lb1_ref, lw2_ref,
                lb2_ref, o_ref, hacc, cacc):
    i = pl.program_id(0)

    @pl.when(i == 0)
    def _():
        hacc[...] = jnp.zeros_like(hacc)
        cacc[...] = jnp.zeros_like(cacc)

    h = s_ref[...] * (p_ref[0] + p_ref[1]) + b_ref[...][None, :]
    h = jnp.maximum(h, 0.0)
    seg = (lax.broadcasted_iota(jnp.int32, (_G, _FBLK), 0)
           == bat_ref[...].reshape(1, _FBLK)).astype(jnp.float32)
    hacc[...] += jnp.dot(seg, h, preferred_element_type=jnp.float32)
    cnt = jnp.sum(seg, axis=1)
    cacc[...] += jnp.broadcast_to(cnt[:, None], (_G, 128))

    @pl.when(i == _NFBLK - 1)
    def _():
        pooled = hacc[...] / jnp.maximum(cacc[...], 1.0)
        z = jnp.maximum(
            jnp.dot(pooled, lw1_ref[...], preferred_element_type=jnp.float32)
            + lb1_ref[...][None, :], 0.0)
        res = jnp.dot(z, lw2_ref[...], preferred_element_type=jnp.float32)
        o_ref[...] = jnp.broadcast_to(res + lb2_ref[...][None, :], (_G, 128))


@jax.jit
def _tc_final(p, scale, b, batch, lw1, lb1, lw2, lb2):
    return pl.pallas_call(
        _final_body,
        grid=(_NFBLK,),
        in_specs=[
            pl.BlockSpec((_NC, _FBLK, _F), lambda i: (0, i, 0)),
            pl.BlockSpec((_FBLK, 1), lambda i: (i, 0)),
            pl.BlockSpec((128,), lambda i: (0,)),
            pl.BlockSpec((_FBLK, 1), lambda i: (i, 0)),
            pl.BlockSpec((128, 64), lambda i: (0, 0)),
            pl.BlockSpec((64,), lambda i: (0,)),
            pl.BlockSpec((64, 1), lambda i: (0, 0)),
            pl.BlockSpec((1,), lambda i: (0,)),
        ],
        out_specs=pl.BlockSpec((_G, 128), lambda i: (0, 0)),
        out_shape=jax.ShapeDtypeStruct((_G, 128), jnp.float32),
        scratch_shapes=[
            pltpu.VMEM((_G, 128), jnp.float32),
            pltpu.VMEM((_G, 128), jnp.float32),
        ],
    )(p, scale, b, batch, lw1, lb1, lw2, lb2)


def kernel(x, hyperedge_index, batch, W1, b1, W2, b2, W3, b3,
           lw1, lb1, lw2, lb2):
    ei0 = hyperedge_index[0]
    ei1 = hyperedge_index[1]

    cnt = _sc_count(ei0, ei1)
    csum = cnt[0] + cnt[1]
    d = csum[0, :_N]
    bb = csum[1, :_HE]
    dinv = jnp.where(d > 0, 1.0 / d, 0.0).reshape(_N, 1)
    binv = jnp.where(bb > 0, 1.0 / bb, 0.0).reshape(_HE, 1)

    xl1 = _tc_mm_plain(x, W1)
    oe1 = _tc_combine(_sc_phase(xl1, ei0, ei1), binv)
    pb1 = _sc_phase(oe1, ei1, ei0)

    xl2a, xl2b = _tc_mm_fused1(pb1, dinv, b1, W2)
    oe2a = _tc_combine(_sc_phase(xl2a, ei0, ei1), binv)
    oe2b = _tc_combine(_sc_phase(xl2b, ei0, ei1), binv)
    pb2a = _sc_phase(oe2a, ei1, ei0)
    pb2b = _sc_phase(oe2b, ei1, ei0)

    xl3 = _tc_mm_fused2(pb2a, pb2b, dinv, b2[:128], b2[128:],
                        W3[:128], W3[128:])
    oe3 = _tc_combine(_sc_phase(xl3, ei0, ei1), binv)
    pb3 = _sc_phase(oe3, ei1, ei0)

    out = _tc_final(pb3, dinv, b3, batch.reshape(_N, 1), lw1, lb1, lw2, lb2)
    return out[:, :1]
